# baseline (speedup 1.0000x reference)
import functools

import jax
import jax.numpy as jnp
import numpy as np
from jax import lax
from jax.experimental import pallas as pl
from jax.experimental.pallas import tpu as pltpu
from jax.experimental.pallas import tpu_sc as plsc

B = 64
D = 10
NPT = 2 ** D - 1
H = 128
E = 128
L = 5
NLEAF = 2 ** (D - 1)
G = 8

NLEAVES = B * NLEAF
SC_NC = 2
SC_NS = 16
SC_NW = SC_NC * SC_NS
CHUNK = 128
NCHUNK = NLEAVES // CHUNK
CPW = NCHUNK // SC_NW


def _emb_body(table_hbm, ids_hbm, x_hbm, idx_buf, rows_buf, x_buf, sem):
    wid = lax.axis_index("s") * SC_NC + lax.axis_index("c")

    def chunk_body(ci, _):
        chunk = wid * CPW + ci
        pltpu.sync_copy(ids_hbm.at[chunk], idx_buf)
        handles = [
            pltpu.async_copy(table_hbm.at[idx_buf.at[g]], rows_buf.at[g], sem)
            for g in range(L)
        ]
        for hnd in handles:
            hnd.wait()

        def row_body(r, _):
            for k in range(E // 16):
                sl = pl.ds(k * 16, 16)
                acc = rows_buf[0, r, sl]
                for g in range(1, L):
                    acc = acc + rows_buf[g, r, sl]
                x_buf[r, sl] = acc
            return _

        lax.fori_loop(0, CHUNK, row_body, None)
        pltpu.sync_copy(x_buf, x_hbm.at[pl.ds(chunk * CHUNK, CHUNK)])
        return _

    lax.fori_loop(0, CPW, chunk_body, None)


def _emb_gather_sum(emb_table, ids_cm):
    mesh = plsc.VectorSubcoreMesh(core_axis_name="c", subcore_axis_name="s")
    return pl.kernel(
        _emb_body,
        out_type=jax.ShapeDtypeStruct((NLEAVES, E), jnp.float32),
        mesh=mesh,
        scratch_types=[
            pltpu.VMEM((L, CHUNK), jnp.int32),
            pltpu.VMEM((L, CHUNK, E), jnp.float32),
            pltpu.VMEM((CHUNK, E), jnp.float32),
            pltpu.SemaphoreType.DMA,
        ],
    )(emb_table, ids_cm)


def _tree_body(x_ref, mask_ref, Wiou_ref, Uiou_ref, biou_ref, Wf_ref, bf_ref,
               h_ref, rooth_ref, rootc_ref, c_ref):
    Wiou = Wiou_ref[...]
    Uiou = Uiou_ref[...]
    biou = biou_ref[...]
    Wf = Wf_ref[...]
    bf = bf_ref[...]

    x = x_ref[...].reshape(G * NLEAF, E) * mask_ref[...]
    iou = jnp.dot(x, Wiou, preferred_element_type=jnp.float32) + biou
    ig = iou[:, :H]
    og = iou[:, H:2 * H]
    ug = iou[:, 2 * H:]
    c_new = jax.nn.sigmoid(ig) * jnp.tanh(ug)
    h_new = jax.nn.sigmoid(og) * jnp.tanh(c_new)
    h_ref[:, NLEAF - 1:NPT, :] = h_new.reshape(G, NLEAF, H)
    c_ref[:, NLEAF - 1:NPT, :] = c_new.reshape(G, NLEAF, H)

    for l in range(D - 2, -1, -1):
        r = 2 ** l
        ps = r - 1
        cs = 2 * r - 1
        hc = h_ref[:, cs:cs + 2 * r, :].reshape(G * 2 * r, H)
        cc = c_ref[:, cs:cs + 2 * r, :].reshape(G * 2 * r, H)
        f = jax.nn.sigmoid(
            lax.dot_general(hc, Wf, (((1,), (1,)), ((), ())),
                            preferred_element_type=jnp.float32) + bf)
        c_in = (f * cc).reshape(G * r, 2, H).sum(axis=1)
        hs = hc.reshape(G * r, 2, H).sum(axis=1)
        iou = jnp.dot(hs, Uiou, preferred_element_type=jnp.float32) + biou
        ig = iou[:, :H]
        og = iou[:, H:2 * H]
        ug = iou[:, 2 * H:]
        c_new = jax.nn.sigmoid(ig) * jnp.tanh(ug) + c_in
        h_new = jax.nn.sigmoid(og) * jnp.tanh(c_new)
        h_ref[:, ps:ps + r, :] = h_new.reshape(G, r, H)
        c_ref[:, ps:ps + r, :] = c_new.reshape(G, r, H)

    rooth_ref[...] = h_new
    rootc_ref[...] = c_new


def _tree_sweep(x_leaf, mask_leaf_f, W_iou, U_iou, b_iou, W_f, b_f2,
                interpret=False):
    grid = (B // G,)
    out_shapes = (
        jax.ShapeDtypeStruct((B, NPT, H), jnp.float32),
        jax.ShapeDtypeStruct((B, H), jnp.float32),
        jax.ShapeDtypeStruct((B, H), jnp.float32),
    )
    return pl.pallas_call(
        _tree_body,
        grid=grid,
        in_specs=[
            pl.BlockSpec((G, NLEAF, E), lambda i: (i, 0, 0)),
            pl.BlockSpec((G * NLEAF, 1), lambda i: (i, 0)),
            pl.BlockSpec((E, 3 * H), lambda i: (0, 0)),
            pl.BlockSpec((H, 3 * H), lambda i: (0, 0)),
            pl.BlockSpec((1, 3 * H), lambda i: (0, 0)),
            pl.BlockSpec((H, H), lambda i: (0, 0)),
            pl.BlockSpec((1, H), lambda i: (0, 0)),
        ],
        out_specs=(
            pl.BlockSpec((G, NPT, H), lambda i: (i, 0, 0)),
            pl.BlockSpec((G, H), lambda i: (i, 0)),
            pl.BlockSpec((G, H), lambda i: (i, 0)),
        ),
        out_shape=out_shapes,
        scratch_shapes=[pltpu.VMEM((G, NPT, H), jnp.float32)],
        interpret=interpret,
    )(x_leaf, mask_leaf_f, W_iou, U_iou, b_iou, W_f, b_f2)


def _impl(wordid, mask, h0, c0, emb_table, W_iou, U_iou, b_iou, W_f, b_f,
          interpret=False):
    ids = (wordid * mask[:, None]).reshape(B, NPT, L)[:, NLEAF - 1:, :]
    mask_leaf = mask.reshape(B, NPT)[:, NLEAF - 1:].astype(
        jnp.float32).reshape(B * NLEAF, 1)
    ids_cm = ids.reshape(NCHUNK, CHUNK, L).transpose(0, 2, 1)
    x_leaf = _emb_gather_sum(emb_table, ids_cm).reshape(B, NLEAF, E)
    tree_output, root_h, root_c = _tree_sweep(
        x_leaf, mask_leaf, W_iou, U_iou, b_iou, W_f,
        b_f.reshape(1, H), interpret=interpret)
    return tree_output, root_h, root_c


def kernel(wordid, mask, h0, c0, emb_table, W_iou, U_iou, b_iou, W_f, b_f):
    return _impl(wordid, mask, h0, c0, emb_table, W_iou, U_iou, b_iou,
                 W_f, b_f)


# speedup vs baseline: 8.0296x; 8.0296x over previous
import functools

import jax
import jax.numpy as jnp
import numpy as np
from jax import lax
from jax.experimental import pallas as pl
from jax.experimental.pallas import tpu as pltpu
from jax.experimental.pallas import tpu_sc as plsc

B = 64
D = 10
NPT = 2 ** D - 1
H = 128
E = 128
L = 5
NLEAF = 2 ** (D - 1)
G = 8

NLEAVES = B * NLEAF
SC_NC = 2
SC_NS = 16
SC_NW = SC_NC * SC_NS
CHUNK = 128
NCHUNK = NLEAVES // CHUNK
CPW = NCHUNK // SC_NW


def _emb_body(table_hbm, ids_hbm, dst_hbm, nsub_hbm, x_hbm,
              idx_buf, dst_buf, nsub_buf, rows_buf, x_buf, sem):
    wid = lax.axis_index("s") * SC_NC + lax.axis_index("c")
    pltpu.sync_copy(nsub_hbm.at[wid], nsub_buf)

    def chunk_body(ci, _):
        chunk = wid * CPW + ci
        pltpu.sync_copy(ids_hbm.at[chunk], idx_buf)
        pltpu.sync_copy(dst_hbm.at[chunk], dst_buf)
        ns = nsub_buf[ci, pl.ds(0, 16)][0]

        def gbody(s, carry):
            idxv = idx_buf[pl.ds(s * 16, 16)]
            pltpu.async_copy(
                table_hbm.at[idxv], rows_buf.at[pl.ds(s * 16, 16)], sem
            ).wait()
            return carry

        lax.fori_loop(0, ns, gbody, None)

        def abody(jj, carry):
            dvec = dst_buf[pl.ds(jj * 16, 16)]
            for i in range(16):
                d = dvec[i]
                base = (jj * 16 + i) * L
                for k in range(E // 16):
                    sl = pl.ds(k * 16, 16)
                    acc = rows_buf[base, sl]
                    for g in range(1, L):
                        acc = acc + rows_buf[base + g, sl]
                    x_buf[d, sl] = acc
            return carry

        lax.fori_loop(0, CHUNK // 16, abody, None)
        pltpu.sync_copy(x_buf, x_hbm.at[pl.ds(chunk * CHUNK, CHUNK)])
        return _

    lax.fori_loop(0, CPW, chunk_body, None)


def _emb_gather_sum(emb_table, ids_cmp, dst, nsub):
    mesh = plsc.VectorSubcoreMesh(core_axis_name="c", subcore_axis_name="s")
    return pl.kernel(
        _emb_body,
        out_type=jax.ShapeDtypeStruct((NLEAVES, E), jnp.float32),
        mesh=mesh,
        scratch_types=[
            pltpu.VMEM((CHUNK * L, ), jnp.int32),
            pltpu.VMEM((CHUNK, ), jnp.int32),
            pltpu.VMEM((CPW, 16), jnp.int32),
            pltpu.VMEM((CHUNK * L, E), jnp.float32),
            pltpu.VMEM((CHUNK, E), jnp.float32),
            pltpu.SemaphoreType.DMA,
        ],
    )(emb_table, ids_cmp, dst, nsub)


def _tree_body(x_ref, mask_ref, Wiou_ref, Uiou_ref, biou_ref, Wf_ref, bf_ref,
               h_ref, rooth_ref, rootc_ref, c_ref):
    Wiou = Wiou_ref[...]
    Uiou = Uiou_ref[...]
    biou = biou_ref[...]
    Wf = Wf_ref[...]
    bf = bf_ref[...]

    x = x_ref[...].reshape(G * NLEAF, E) * mask_ref[...]
    iou = jnp.dot(x, Wiou, preferred_element_type=jnp.float32) + biou
    ig = iou[:, :H]
    og = iou[:, H:2 * H]
    ug = iou[:, 2 * H:]
    c_new = jax.nn.sigmoid(ig) * jnp.tanh(ug)
    h_new = jax.nn.sigmoid(og) * jnp.tanh(c_new)
    h_ref[:, NLEAF - 1:NPT, :] = h_new.reshape(G, NLEAF, H)
    c_ref[:, NLEAF - 1:NPT, :] = c_new.reshape(G, NLEAF, H)

    for l in range(D - 2, -1, -1):
        r = 2 ** l
        ps = r - 1
        cs = 2 * r - 1
        hc = h_ref[:, cs:cs + 2 * r, :].reshape(G * 2 * r, H)
        cc = c_ref[:, cs:cs + 2 * r, :].reshape(G * 2 * r, H)
        f = jax.nn.sigmoid(
            lax.dot_general(hc, Wf, (((1,), (1,)), ((), ())),
                            preferred_element_type=jnp.float32) + bf)
        c_in = (f * cc).reshape(G * r, 2, H).sum(axis=1)
        hs = hc.reshape(G * r, 2, H).sum(axis=1)
        iou = jnp.dot(hs, Uiou, preferred_element_type=jnp.float32) + biou
        ig = iou[:, :H]
        og = iou[:, H:2 * H]
        ug = iou[:, 2 * H:]
        c_new = jax.nn.sigmoid(ig) * jnp.tanh(ug) + c_in
        h_new = jax.nn.sigmoid(og) * jnp.tanh(c_new)
        h_ref[:, ps:ps + r, :] = h_new.reshape(G, r, H)
        c_ref[:, ps:ps + r, :] = c_new.reshape(G, r, H)

    rooth_ref[...] = h_new
    rootc_ref[...] = c_new


def _tree_sweep(x_leaf, mask_leaf_f, W_iou, U_iou, b_iou, W_f, b_f2,
                interpret=False):
    grid = (B // G,)
    out_shapes = (
        jax.ShapeDtypeStruct((B, NPT, H), jnp.float32),
        jax.ShapeDtypeStruct((B, H), jnp.float32),
        jax.ShapeDtypeStruct((B, H), jnp.float32),
    )
    return pl.pallas_call(
        _tree_body,
        grid=grid,
        in_specs=[
            pl.BlockSpec((G, NLEAF, E), lambda i: (i, 0, 0)),
            pl.BlockSpec((G * NLEAF, 1), lambda i: (i, 0)),
            pl.BlockSpec((E, 3 * H), lambda i: (0, 0)),
            pl.BlockSpec((H, 3 * H), lambda i: (0, 0)),
            pl.BlockSpec((1, 3 * H), lambda i: (0, 0)),
            pl.BlockSpec((H, H), lambda i: (0, 0)),
            pl.BlockSpec((1, H), lambda i: (0, 0)),
        ],
        out_specs=(
            pl.BlockSpec((G, NPT, H), lambda i: (i, 0, 0)),
            pl.BlockSpec((G, H), lambda i: (i, 0)),
            pl.BlockSpec((G, H), lambda i: (i, 0)),
        ),
        out_shape=out_shapes,
        scratch_shapes=[pltpu.VMEM((G, NPT, H), jnp.float32)],
        interpret=interpret,
    )(x_leaf, mask_leaf_f, W_iou, U_iou, b_iou, W_f, b_f2)


def _impl(wordid, mask, h0, c0, emb_table, W_iou, U_iou, b_iou, W_f, b_f,
          interpret=False):
    ids = wordid.reshape(B, NPT, L)[:, NLEAF - 1:, :]
    m = mask.reshape(B, NPT)[:, NLEAF - 1:].reshape(NCHUNK, CHUNK)
    mask_leaf = m.astype(jnp.float32).reshape(B * NLEAF, 1)
    order = jnp.argsort(1 - m, axis=1, stable=True).astype(jnp.int32)
    ids_sorted = jnp.take_along_axis(
        ids.reshape(NCHUNK, CHUNK, L), order[:, :, None], axis=1)
    ids_cmp = ids_sorted.reshape(NCHUNK, CHUNK * L)
    cnt = jnp.sum(m, axis=1)
    nsub = ((cnt * L + 15) // 16).astype(jnp.int32)
    nsub_b = jnp.broadcast_to(
        nsub.reshape(SC_NW, CPW, 1), (SC_NW, CPW, 16)).astype(jnp.int32)
    x_leaf = _emb_gather_sum(
        emb_table, ids_cmp, order, nsub_b).reshape(B, NLEAF, E)
    tree_output, root_h, root_c = _tree_sweep(
        x_leaf, mask_leaf, W_iou, U_iou, b_iou, W_f,
        b_f.reshape(1, H), interpret=interpret)
    return tree_output, root_h, root_c


def kernel(wordid, mask, h0, c0, emb_table, W_iou, U_iou, b_iou, W_f, b_f):
    return _impl(wordid, mask, h0, c0, emb_table, W_iou, U_iou, b_iou,
                 W_f, b_f)


# speedup vs baseline: 8.4317x; 1.0501x over previous
import functools

import jax
import jax.numpy as jnp
import numpy as np
from jax import lax
from jax.experimental import pallas as pl
from jax.experimental.pallas import tpu as pltpu
from jax.experimental.pallas import tpu_sc as plsc

B = 64
D = 10
NPT = 2 ** D - 1
H = 128
E = 128
L = 5
NLEAF = 2 ** (D - 1)
G = 8

NLEAVES = B * NLEAF
SC_NC = 2
SC_NS = 16
SC_NW = SC_NC * SC_NS
CHUNK = 128
NCHUNK = NLEAVES // CHUNK
CPW = NCHUNK // SC_NW


def _emb_body(table_hbm, ids_hbm, dst_hbm, nsub_hbm, x_hbm,
              idx_buf, dst_buf, nsub_buf, rows_buf, x_buf, sem):
    wid = lax.axis_index("s") * SC_NC + lax.axis_index("c")
    pltpu.sync_copy(nsub_hbm.at[wid], nsub_buf)

    def chunk_body(ci, _):
        chunk = wid * CPW + ci
        pltpu.sync_copy(ids_hbm.at[chunk], idx_buf)
        pltpu.sync_copy(dst_hbm.at[chunk], dst_buf)
        ns = nsub_buf[ci, pl.ds(0, 16)][0]

        def gbody(s, carry):
            idxv = idx_buf[pl.ds(s * 16, 16)]
            pltpu.async_copy(
                table_hbm.at[idxv], rows_buf.at[pl.ds(s * 16, 16)], sem
            ).wait()
            return carry

        lax.fori_loop(0, ns, gbody, None)

        def abody(jj, carry):
            dvec = dst_buf[pl.ds(jj * 16, 16)]
            for i in range(16):
                d = dvec[i]
                base = (jj * 16 + i) * L
                for k in range(E // 16):
                    sl = pl.ds(k * 16, 16)
                    acc = rows_buf[base, sl]
                    for g in range(1, L):
                        acc = acc + rows_buf[base + g, sl]
                    x_buf[d, sl] = acc
            return carry

        lax.fori_loop(0, CHUNK // 16, abody, None)
        pltpu.sync_copy(x_buf, x_hbm.at[pl.ds(chunk * CHUNK, CHUNK)])
        return _

    lax.fori_loop(0, CPW, chunk_body, None)


def _emb_gather_sum(emb_table, ids_cmp, dst, nsub):
    mesh = plsc.VectorSubcoreMesh(core_axis_name="c", subcore_axis_name="s")
    return pl.kernel(
        _emb_body,
        out_type=jax.ShapeDtypeStruct((NLEAVES, E), jnp.float32),
        mesh=mesh,
        scratch_types=[
            pltpu.VMEM((CHUNK * L, ), jnp.int32),
            pltpu.VMEM((CHUNK, ), jnp.int32),
            pltpu.VMEM((CPW, 16), jnp.int32),
            pltpu.VMEM((CHUNK * L, E), jnp.float32),
            pltpu.VMEM((CHUNK, E), jnp.float32),
            pltpu.SemaphoreType.DMA,
        ],
    )(emb_table, ids_cmp, dst, nsub)


def _tree_body(x_ref, mask_ref, Wiou_ref, Uiou_ref, biou_ref, Wf_ref, bf_ref,
               h_ref, rooth_ref, rootc_ref, hs_ref, cs_ref):
    Wiou = Wiou_ref[...]
    Uiou = Uiou_ref[...]
    biou = biou_ref[...]
    Wf = Wf_ref[...]
    bf = bf_ref[...]

    x = x_ref[...].reshape(G * NLEAF, E) * mask_ref[...]
    iou = jnp.dot(x, Wiou, preferred_element_type=jnp.float32) + biou
    ig = iou[:, :H]
    og = iou[:, H:2 * H]
    ug = iou[:, 2 * H:]
    c_new = jax.nn.sigmoid(ig) * jnp.tanh(ug)
    h_new = jax.nn.sigmoid(og) * jnp.tanh(c_new)
    h_ref[:, NLEAF - 1:NPT, :] = h_new.reshape(G, NLEAF, H)
    hs_ref[:, NLEAF // 2:NLEAF, :, :] = h_new.reshape(G, NLEAF // 2, 2, H)
    cs_ref[:, NLEAF // 2:NLEAF, :, :] = c_new.reshape(G, NLEAF // 2, 2, H)

    for l in range(D - 2, -1, -1):
        r = 2 ** l
        ps = r - 1
        h_l = hs_ref[:, r:2 * r, 0, :].reshape(G * r, H)
        h_r = hs_ref[:, r:2 * r, 1, :].reshape(G * r, H)
        c_l = cs_ref[:, r:2 * r, 0, :].reshape(G * r, H)
        c_r = cs_ref[:, r:2 * r, 1, :].reshape(G * r, H)
        f_l = jax.nn.sigmoid(
            lax.dot_general(h_l, Wf, (((1,), (1,)), ((), ())),
                            preferred_element_type=jnp.float32) + bf)
        f_r = jax.nn.sigmoid(
            lax.dot_general(h_r, Wf, (((1,), (1,)), ((), ())),
                            preferred_element_type=jnp.float32) + bf)
        c_in = f_l * c_l + f_r * c_r
        hsum = h_l + h_r
        iou = jnp.dot(hsum, Uiou, preferred_element_type=jnp.float32) + biou
        ig = iou[:, :H]
        og = iou[:, H:2 * H]
        ug = iou[:, 2 * H:]
        c_new = jax.nn.sigmoid(ig) * jnp.tanh(ug) + c_in
        h_new = jax.nn.sigmoid(og) * jnp.tanh(c_new)
        h_ref[:, ps:ps + r, :] = h_new.reshape(G, r, H)
        if r >= 2:
            hs_ref[:, r // 2:r, :, :] = h_new.reshape(G, r // 2, 2, H)
            cs_ref[:, r // 2:r, :, :] = c_new.reshape(G, r // 2, 2, H)

    rooth_ref[...] = h_new
    rootc_ref[...] = c_new


def _tree_sweep(x_leaf, mask_leaf_f, W_iou, U_iou, b_iou, W_f, b_f2,
                interpret=False):
    grid = (B // G,)
    out_shapes = (
        jax.ShapeDtypeStruct((B, NPT, H), jnp.float32),
        jax.ShapeDtypeStruct((B, H), jnp.float32),
        jax.ShapeDtypeStruct((B, H), jnp.float32),
    )
    return pl.pallas_call(
        _tree_body,
        grid=grid,
        in_specs=[
            pl.BlockSpec((G, NLEAF, E), lambda i: (i, 0, 0)),
            pl.BlockSpec((G * NLEAF, 1), lambda i: (i, 0)),
            pl.BlockSpec((E, 3 * H), lambda i: (0, 0)),
            pl.BlockSpec((H, 3 * H), lambda i: (0, 0)),
            pl.BlockSpec((1, 3 * H), lambda i: (0, 0)),
            pl.BlockSpec((H, H), lambda i: (0, 0)),
            pl.BlockSpec((1, H), lambda i: (0, 0)),
        ],
        out_specs=(
            pl.BlockSpec((G, NPT, H), lambda i: (i, 0, 0)),
            pl.BlockSpec((G, H), lambda i: (i, 0)),
            pl.BlockSpec((G, H), lambda i: (i, 0)),
        ),
        out_shape=out_shapes,
        scratch_shapes=[pltpu.VMEM((G, NPT // 2 + 1, 2, H), jnp.float32),
                        pltpu.VMEM((G, NPT // 2 + 1, 2, H), jnp.float32)],
        interpret=interpret,
    )(x_leaf, mask_leaf_f, W_iou, U_iou, b_iou, W_f, b_f2)


def _impl(wordid, mask, h0, c0, emb_table, W_iou, U_iou, b_iou, W_f, b_f,
          interpret=False):
    ids = wordid.reshape(B, NPT, L)[:, NLEAF - 1:, :]
    m = mask.reshape(B, NPT)[:, NLEAF - 1:].reshape(NCHUNK, CHUNK)
    mask_leaf = m.astype(jnp.float32).reshape(B * NLEAF, 1)
    order = jnp.argsort(1 - m, axis=1, stable=True).astype(jnp.int32)
    ids_sorted = jnp.take_along_axis(
        ids.reshape(NCHUNK, CHUNK, L), order[:, :, None], axis=1)
    ids_cmp = ids_sorted.reshape(NCHUNK, CHUNK * L)
    cnt = jnp.sum(m, axis=1)
    nsub = ((cnt * L + 15) // 16).astype(jnp.int32)
    nsub_b = jnp.broadcast_to(
        nsub.reshape(SC_NW, CPW, 1), (SC_NW, CPW, 16)).astype(jnp.int32)
    x_leaf = _emb_gather_sum(
        emb_table, ids_cmp, order, nsub_b).reshape(B, NLEAF, E)
    tree_output, root_h, root_c = _tree_sweep(
        x_leaf, mask_leaf, W_iou, U_iou, b_iou, W_f,
        b_f.reshape(1, H), interpret=interpret)
    return tree_output, root_h, root_c


def kernel(wordid, mask, h0, c0, emb_table, W_iou, U_iou, b_iou, W_f, b_f):
    return _impl(wordid, mask, h0, c0, emb_table, W_iou, U_iou, b_iou,
                 W_f, b_f)
